# Initial kernel scaffold; baseline (speedup 1.0000x reference)
import jax
import jax.numpy as jnp
from jax.experimental import pallas as pl


def kernel(jd, resume, position, job_emb, geek_emb, W1, W21a, W21b, W22a, W22b, W23a, W23b):
    raise NotImplementedError("write your pallas kernel here")


# speedup vs baseline: 3.2532x; 3.2532x over previous
import functools

import jax
import jax.numpy as jnp
from jax import lax
from jax.experimental import pallas as pl
from jax.experimental.pallas import tpu as pltpu
from jax.experimental.pallas import tpu_sc as plsc

B, L = 4096, 50
EMB = 96
DIM = EMB * 2

NC, NS = 2, 16
NW = NC * NS
BPW = B // NW
G = 8
CHUNKS = BPW // G
ROWS = G * L
SUB = 80
NSUB = ROWS // SUB
CH = EMB // 16


def _bag_pool_sc(jd_flat, rs_flat, job_emb, geek_emb):
    mesh = plsc.VectorSubcoreMesh(core_axis_name="c", subcore_axis_name="s")

    @functools.partial(
        pl.kernel,
        mesh=mesh,
        out_type=jax.ShapeDtypeStruct((B, DIM), jnp.float32),
        scratch_types=[
            pltpu.VMEM((ROWS,), jnp.int32),
            pltpu.VMEM((ROWS,), jnp.int32),
            pltpu.VMEM((ROWS, EMB), jnp.float32),
            pltpu.VMEM((ROWS, EMB), jnp.float32),
            pltpu.VMEM((G, DIM), jnp.float32),
            pltpu.SemaphoreType.DMA,
        ],
        compiler_params=pltpu.CompilerParams(use_tc_tiling_on_sc=False),
    )
    def kern(jd_hbm, rs_hbm, job_hbm, geek_hbm, out_hbm,
             jd_idx, rs_idx, jd_rows, rs_rows, out_v, sem):
        w = lax.axis_index("s") * NC + lax.axis_index("c")

        def accum_body(j, _):
            base = j * L
            for t, rows in ((0, jd_rows), (1, rs_rows)):
                accs = [rows[base, pl.ds(c * 16, 16)] for c in range(CH)]
                for r in range(1, L):
                    for c in range(CH):
                        accs[c] = accs[c] + rows[base + r, pl.ds(c * 16, 16)]
                for c in range(CH):
                    out_v[j, pl.ds(t * EMB + c * 16, 16)] = accs[c]
            return ()

        def chunk_body(k, _):
            bag_base = w * BPW + k * G
            idx_off = bag_base * L
            pltpu.sync_copy(jd_hbm.at[pl.ds(idx_off, ROWS)], jd_idx)
            pltpu.sync_copy(rs_hbm.at[pl.ds(idx_off, ROWS)], rs_idx)
            cps = []
            for s in range(NSUB):
                sl = pl.ds(s * SUB, SUB)
                cps.append(pltpu.async_copy(
                    job_hbm.at[jd_idx.at[sl]], jd_rows.at[sl], sem))
                cps.append(pltpu.async_copy(
                    geek_hbm.at[rs_idx.at[sl]], rs_rows.at[sl], sem))
            for cp in cps:
                cp.wait()
            lax.fori_loop(0, G, accum_body, (), unroll=False)
            pltpu.sync_copy(out_v, out_hbm.at[pl.ds(bag_base, G)])
            return ()

        lax.fori_loop(0, CHUNKS, chunk_body, (), unroll=False)

    return kern(jd_flat, rs_flat, job_emb, geek_emb)


def _mlp_body(x_ref, w1_ref, w2a_ref, w2b_ref, o_ref):
    x = x_ref[...]
    h = jnp.maximum(
        jnp.dot(x, w1_ref[...], preferred_element_type=jnp.float32), 0.0)
    h2 = jnp.maximum(
        jnp.dot(h, w2a_ref[...], preferred_element_type=jnp.float32), 0.0)
    z = jnp.dot(h2, w2b_ref[...], preferred_element_type=jnp.float32)
    o_ref[...] = 1.0 / (1.0 + jnp.exp(-z))


def _mlp_tc(x, w1t, w2at, w2bt):
    return pl.pallas_call(
        _mlp_body,
        out_shape=jax.ShapeDtypeStruct((B, 3), jnp.float32),
    )(x, w1t, w2at, w2bt)


def kernel(jd, resume, position, job_emb, geek_emb,
           W1, W21a, W21b, W22a, W22b, W23a, W23b):
    del position
    jd_flat = jnp.reshape(jd, (B * L,))
    rs_flat = jnp.reshape(resume, (B * L,))

    x = _bag_pool_sc(jd_flat, rs_flat, job_emb, geek_emb)

    H = DIM // 6
    w2a = jnp.zeros((EMB, EMB), jnp.float32)
    w2a = w2a.at[0:H, 0:64].set(W21a)
    w2a = w2a.at[H:2 * H, 32:96].set(W22a)
    w2a = w2a.at[2 * H:3 * H, :].set(W23a)
    w2b = jnp.zeros((3, EMB), jnp.float32)
    w2b = w2b.at[0, 0:H].set(W21b[0])
    w2b = w2b.at[1, H:2 * H].set(W22b[0])
    w2b = w2b.at[2, 2 * H:3 * H].set(W23b[0])

    return _mlp_tc(x, W1.T, w2a.T, w2b.T)


# speedup vs baseline: 3.2772x; 1.0074x over previous
import functools

import jax
import jax.numpy as jnp
from jax import lax
from jax.experimental import pallas as pl
from jax.experimental.pallas import tpu as pltpu
from jax.experimental.pallas import tpu_sc as plsc

B, L = 4096, 50
EMB = 96
DIM = EMB * 2

EMB_P = 128
NC, NS = 2, 16
NW = NC * NS
BPW = B // NW
G = 8
CHUNKS = BPW // G
ROWS = G * L
SUB = 80
NSUB = ROWS // SUB
CH = EMB // 16


def _bag_pool_sc(jd_flat, rs_flat, job_emb, geek_emb):
    mesh = plsc.VectorSubcoreMesh(core_axis_name="c", subcore_axis_name="s")

    @functools.partial(
        pl.kernel,
        mesh=mesh,
        out_type=jax.ShapeDtypeStruct((B, DIM), jnp.float32),
        scratch_types=[
            pltpu.VMEM((ROWS,), jnp.int32),
            pltpu.VMEM((ROWS,), jnp.int32),
            pltpu.VMEM((ROWS, EMB_P), jnp.float32),
            pltpu.VMEM((ROWS, EMB_P), jnp.float32),
            pltpu.VMEM((G, DIM), jnp.float32),
            pltpu.SemaphoreType.DMA,
        ],
    )
    def kern(jd_hbm, rs_hbm, job_hbm, geek_hbm, out_hbm,
             jd_idx, rs_idx, jd_rows, rs_rows, out_v, sem):
        w = lax.axis_index("s") * NC + lax.axis_index("c")

        def accum_body(j, _):
            base = j * L
            for t, rows in ((0, jd_rows), (1, rs_rows)):
                accs = [rows[base, pl.ds(c * 16, 16)] for c in range(CH)]
                for r in range(1, L):
                    for c in range(CH):
                        accs[c] = accs[c] + rows[base + r, pl.ds(c * 16, 16)]
                for c in range(CH):
                    out_v[j, pl.ds(t * EMB + c * 16, 16)] = accs[c]
            return ()

        def chunk_body(k, _):
            bag_base = w * BPW + k * G
            idx_off = bag_base * L
            pltpu.sync_copy(jd_hbm.at[pl.ds(idx_off, ROWS)], jd_idx)
            pltpu.sync_copy(rs_hbm.at[pl.ds(idx_off, ROWS)], rs_idx)
            cps = []
            for s in range(NSUB):
                sl = pl.ds(s * SUB, SUB)
                cps.append(pltpu.async_copy(
                    job_hbm.at[jd_idx.at[sl]], jd_rows.at[sl], sem))
                cps.append(pltpu.async_copy(
                    geek_hbm.at[rs_idx.at[sl]], rs_rows.at[sl], sem))
            for cp in cps:
                cp.wait()
            lax.fori_loop(0, G, accum_body, (), unroll=False)
            pltpu.sync_copy(out_v, out_hbm.at[pl.ds(bag_base, G)])
            return ()

        lax.fori_loop(0, CHUNKS, chunk_body, (), unroll=False)

    return kern(jd_flat, rs_flat, job_emb, geek_emb)


def _mlp_body(x_ref, w1_ref, w2a_ref, w2b_ref, o_ref):
    x = x_ref[...]
    h = jnp.maximum(
        jnp.dot(x, w1_ref[...], preferred_element_type=jnp.float32), 0.0)
    h2 = jnp.maximum(
        jnp.dot(h, w2a_ref[...], preferred_element_type=jnp.float32), 0.0)
    z = jnp.dot(h2, w2b_ref[...], preferred_element_type=jnp.float32)
    o_ref[...] = 1.0 / (1.0 + jnp.exp(-z))


def _mlp_tc(x, w1t, w2at, w2bt):
    return pl.pallas_call(
        _mlp_body,
        out_shape=jax.ShapeDtypeStruct((B, 3), jnp.float32),
    )(x, w1t, w2at, w2bt)


def kernel(jd, resume, position, job_emb, geek_emb,
           W1, W21a, W21b, W22a, W22b, W23a, W23b):
    del position
    jd_flat = jnp.reshape(jd, (B * L,))
    rs_flat = jnp.reshape(resume, (B * L,))
    job_pad = jnp.pad(job_emb, ((0, 0), (0, EMB_P - EMB)))
    geek_pad = jnp.pad(geek_emb, ((0, 0), (0, EMB_P - EMB)))

    x = _bag_pool_sc(jd_flat, rs_flat, job_pad, geek_pad)

    H = DIM // 6
    w2a = jnp.zeros((EMB, EMB), jnp.float32)
    w2a = w2a.at[0:H, 0:64].set(W21a)
    w2a = w2a.at[H:2 * H, 32:96].set(W22a)
    w2a = w2a.at[2 * H:3 * H, :].set(W23a)
    w2b = jnp.zeros((3, EMB), jnp.float32)
    w2b = w2b.at[0, 0:H].set(W21b[0])
    w2b = w2b.at[1, H:2 * H].set(W22b[0])
    w2b = w2b.at[2, 2 * H:3 * H].set(W23b[0])

    return _mlp_tc(x, W1.T, w2a.T, w2b.T)


# speedup vs baseline: 5.0863x; 1.5520x over previous
import functools

import jax
import jax.numpy as jnp
from jax import lax
from jax.experimental import pallas as pl
from jax.experimental.pallas import tpu as pltpu
from jax.experimental.pallas import tpu_sc as plsc

B, L = 4096, 50
EMB = 96
DIM = EMB * 2

EMB_P = 128
NC, NS = 2, 16
NW = NC * NS
BPW = B // NW
G = 8
CHUNKS = BPW // G
ROWS = G * L
SUB = 80
NSUB = ROWS // SUB
CH = EMB // 16


def _bag_pool_sc(jd_flat, rs_flat, job_emb, geek_emb):
    mesh = plsc.VectorSubcoreMesh(core_axis_name="c", subcore_axis_name="s")

    @functools.partial(
        pl.kernel,
        mesh=mesh,
        out_type=jax.ShapeDtypeStruct((B, DIM), jnp.float32),
        scratch_types=[
            pltpu.VMEM((ROWS,), jnp.int32),
            pltpu.VMEM((ROWS,), jnp.int32),
            pltpu.VMEM((ROWS, EMB_P), jnp.float32),
            pltpu.VMEM((ROWS, EMB_P), jnp.float32),
            pltpu.VMEM((G, DIM), jnp.float32),
            pltpu.SemaphoreType.DMA,
        ],
    )
    def kern(jd_hbm, rs_hbm, job_hbm, geek_hbm, out_hbm,
             jd_idx, rs_idx, jd_rows, rs_rows, out_v, sem):
        w = lax.axis_index("s") * NC + lax.axis_index("c")

        def accum_body(j, _):
            base = j * L
            for t, rows in ((0, jd_rows), (1, rs_rows)):
                accs = [rows[base, pl.ds(c * 16, 16)] for c in range(CH)]
                for r in range(1, L):
                    for c in range(CH):
                        accs[c] = accs[c] + rows[base + r, pl.ds(c * 16, 16)]
                for c in range(CH):
                    out_v[j, pl.ds(t * EMB + c * 16, 16)] = accs[c]
            return ()

        def chunk_body(k, _):
            bag_base = w * BPW + k * G
            idx_off = bag_base * L
            pltpu.sync_copy(jd_hbm.at[pl.ds(idx_off, ROWS)], jd_idx)
            pltpu.sync_copy(rs_hbm.at[pl.ds(idx_off, ROWS)], rs_idx)
            cps = []
            for s in range(NSUB):
                sl = pl.ds(s * SUB, SUB)
                cps.append(pltpu.async_copy(
                    job_hbm.at[jd_idx.at[sl]], jd_rows.at[sl], sem))
                cps.append(pltpu.async_copy(
                    geek_hbm.at[rs_idx.at[sl]], rs_rows.at[sl], sem))
            for cp in cps:
                cp.wait()
            lax.fori_loop(0, G, accum_body, (), unroll=False)
            pltpu.sync_copy(out_v, out_hbm.at[pl.ds(bag_base, G)])
            return ()

        lax.fori_loop(0, CHUNKS, chunk_body, (), unroll=False)

    return kern(jd_flat, rs_flat, job_emb, geek_emb)


def _pad_body(a_ref, b_ref, oa_ref, ob_ref):
    pad = ((0, 0), (0, EMB_P - EMB))
    oa_ref[...] = jnp.pad(a_ref[...], pad)
    ob_ref[...] = jnp.pad(b_ref[...], pad)


def _pad_tables_tc(job_emb, geek_emb):
    n = job_emb.shape[0]
    bm = 10000
    grid = n // bm
    return pl.pallas_call(
        _pad_body,
        grid=(grid,),
        in_specs=[
            pl.BlockSpec((bm, EMB), lambda i: (i, 0)),
            pl.BlockSpec((bm, EMB), lambda i: (i, 0)),
        ],
        out_specs=[
            pl.BlockSpec((bm, EMB_P), lambda i: (i, 0)),
            pl.BlockSpec((bm, EMB_P), lambda i: (i, 0)),
        ],
        out_shape=[
            jax.ShapeDtypeStruct((n, EMB_P), jnp.float32),
            jax.ShapeDtypeStruct((n, EMB_P), jnp.float32),
        ],
    )(job_emb, geek_emb)


def _mlp_body(x_ref, w1_ref, w2a_ref, w2b_ref, o_ref):
    x = x_ref[...]
    h = jnp.maximum(
        jnp.dot(x, w1_ref[...], preferred_element_type=jnp.float32), 0.0)
    h2 = jnp.maximum(
        jnp.dot(h, w2a_ref[...], preferred_element_type=jnp.float32), 0.0)
    z = jnp.dot(h2, w2b_ref[...], preferred_element_type=jnp.float32)
    o_ref[...] = 1.0 / (1.0 + jnp.exp(-z))


def _mlp_tc(x, w1t, w2at, w2bt):
    return pl.pallas_call(
        _mlp_body,
        out_shape=jax.ShapeDtypeStruct((B, 3), jnp.float32),
    )(x, w1t, w2at, w2bt)


def kernel(jd, resume, position, job_emb, geek_emb,
           W1, W21a, W21b, W22a, W22b, W23a, W23b):
    del position
    jd_flat = jnp.reshape(jd, (B * L,))
    rs_flat = jnp.reshape(resume, (B * L,))
    job_pad, geek_pad = _pad_tables_tc(job_emb, geek_emb)

    x = _bag_pool_sc(jd_flat, rs_flat, job_pad, geek_pad)

    H = DIM // 6
    w2a = jnp.zeros((EMB, EMB), jnp.float32)
    w2a = w2a.at[0:H, 0:64].set(W21a)
    w2a = w2a.at[H:2 * H, 32:96].set(W22a)
    w2a = w2a.at[2 * H:3 * H, :].set(W23a)
    w2b = jnp.zeros((3, EMB), jnp.float32)
    w2b = w2b.at[0, 0:H].set(W21b[0])
    w2b = w2b.at[1, H:2 * H].set(W22b[0])
    w2b = w2b.at[2, 2 * H:3 * H].set(W23b[0])

    return _mlp_tc(x, W1.T, w2a.T, w2b.T)


# speedup vs baseline: 6.0453x; 1.1886x over previous
import functools

import jax
import jax.numpy as jnp
from jax import lax
from jax.experimental import pallas as pl
from jax.experimental.pallas import tpu as pltpu
from jax.experimental.pallas import tpu_sc as plsc

B, L = 4096, 50
EMB = 96
DIM = EMB * 2

EMB_P = 128
NC, NS = 2, 16
NW = NC * NS
BPW = B // NW
G = 8
CHUNKS = BPW // G
ROWS = G * L
SUB = 80
NSUB = ROWS // SUB
CH = EMB // 16


def _bag_pool_sc(idx_flat, table):
    mesh = plsc.VectorSubcoreMesh(core_axis_name="c", subcore_axis_name="s")

    @functools.partial(
        pl.kernel,
        mesh=mesh,
        out_type=jax.ShapeDtypeStruct((B, EMB), jnp.float32),
        scratch_types=[
            pltpu.VMEM((ROWS,), jnp.int32),
            pltpu.VMEM((ROWS,), jnp.int32),
            pltpu.VMEM((ROWS, EMB_P), jnp.float32),
            pltpu.VMEM((ROWS, EMB_P), jnp.float32),
            pltpu.VMEM((BPW, EMB), jnp.float32),
            pltpu.SemaphoreType.DMA,
            pltpu.SemaphoreType.DMA,
        ],
    )
    def kern(idx_hbm, tab_hbm, out_hbm, idx0, idx1, buf0, buf1, out_acc,
             sem0, sem1):
        w = lax.axis_index("s") * NC + lax.axis_index("c")
        base_i = w * (BPW * L)

        def fire(k, idxbuf, rowbuf, sem):
            pltpu.sync_copy(idx_hbm.at[pl.ds(base_i + k * ROWS, ROWS)],
                            idxbuf)
            for s in range(NSUB):
                sl = pl.ds(s * SUB, SUB)
                pltpu.async_copy(tab_hbm.at[idxbuf.at[sl]], rowbuf.at[sl],
                                 sem)

        def drain(rowbuf, sem):
            pltpu.make_async_copy(tab_hbm.at[pl.ds(0, ROWS)], rowbuf,
                                  sem).wait()

        def accum(k, rowbuf):
            def bag(j, _):
                base = j * L
                accs = [rowbuf[base, pl.ds(c * 16, 16)] for c in range(CH)]
                for r in range(1, L):
                    for c in range(CH):
                        accs[c] = accs[c] + rowbuf[base + r,
                                                   pl.ds(c * 16, 16)]
                row = k * G + j
                for c in range(CH):
                    out_acc[row, pl.ds(c * 16, 16)] = accs[c]
                return ()
            lax.fori_loop(0, G, bag, (), unroll=False)

        fire(0, idx0, buf0, sem0)
        fire(1, idx1, buf1, sem1)

        def body(m, _):
            k = 2 * m
            drain(buf0, sem0)
            accum(k, buf0)
            fire(k + 2, idx0, buf0, sem0)
            drain(buf1, sem1)
            accum(k + 1, buf1)
            fire(k + 3, idx1, buf1, sem1)
            return ()

        lax.fori_loop(0, CHUNKS // 2 - 1, body, (), unroll=False)
        drain(buf0, sem0)
        accum(CHUNKS - 2, buf0)
        drain(buf1, sem1)
        accum(CHUNKS - 1, buf1)
        pltpu.sync_copy(out_acc, out_hbm.at[pl.ds(w * BPW, BPW)])

    return kern(idx_flat, table)


def _pad_body(a_ref, b_ref, oa_ref, ob_ref):
    pad = ((0, 0), (0, EMB_P - EMB))
    oa_ref[...] = jnp.pad(a_ref[...], pad)
    ob_ref[...] = jnp.pad(b_ref[...], pad)


def _pad_tables_tc(job_emb, geek_emb):
    n = job_emb.shape[0]
    bm = 10000
    grid = n // bm
    return pl.pallas_call(
        _pad_body,
        grid=(grid,),
        in_specs=[
            pl.BlockSpec((bm, EMB), lambda i: (i, 0)),
            pl.BlockSpec((bm, EMB), lambda i: (i, 0)),
        ],
        out_specs=[
            pl.BlockSpec((bm, EMB_P), lambda i: (i, 0)),
            pl.BlockSpec((bm, EMB_P), lambda i: (i, 0)),
        ],
        out_shape=[
            jax.ShapeDtypeStruct((n, EMB_P), jnp.float32),
            jax.ShapeDtypeStruct((n, EMB_P), jnp.float32),
        ],
    )(job_emb, geek_emb)


def _mlp_body(xj_ref, xr_ref, w1a_ref, w1b_ref, w2a_ref, w2b_ref, o_ref):
    h = jnp.dot(xj_ref[...], w1a_ref[...],
                preferred_element_type=jnp.float32)
    h += jnp.dot(xr_ref[...], w1b_ref[...],
                 preferred_element_type=jnp.float32)
    h = jnp.maximum(h, 0.0)
    h2 = jnp.maximum(
        jnp.dot(h, w2a_ref[...], preferred_element_type=jnp.float32), 0.0)
    z = jnp.dot(h2, w2b_ref[...], preferred_element_type=jnp.float32)
    o_ref[...] = 1.0 / (1.0 + jnp.exp(-z))


def _mlp_tc(xj, xr, w1at, w1bt, w2at, w2bt):
    return pl.pallas_call(
        _mlp_body,
        out_shape=jax.ShapeDtypeStruct((B, 3), jnp.float32),
    )(xj, xr, w1at, w1bt, w2at, w2bt)


def kernel(jd, resume, position, job_emb, geek_emb,
           W1, W21a, W21b, W22a, W22b, W23a, W23b):
    del position
    jd_flat = jnp.reshape(jd, (B * L,))
    rs_flat = jnp.reshape(resume, (B * L,))
    job_pad, geek_pad = _pad_tables_tc(job_emb, geek_emb)

    xj = _bag_pool_sc(jd_flat, job_pad)
    xr = _bag_pool_sc(rs_flat, geek_pad)

    H = DIM // 6
    w2a = jnp.zeros((EMB, EMB), jnp.float32)
    w2a = w2a.at[0:H, 0:64].set(W21a)
    w2a = w2a.at[H:2 * H, 32:96].set(W22a)
    w2a = w2a.at[2 * H:3 * H, :].set(W23a)
    w2b = jnp.zeros((3, EMB), jnp.float32)
    w2b = w2b.at[0, 0:H].set(W21b[0])
    w2b = w2b.at[1, H:2 * H].set(W22b[0])
    w2b = w2b.at[2, 2 * H:3 * H].set(W23b[0])

    w1t = W1.T
    return _mlp_tc(xj, xr, w1t[:EMB], w1t[EMB:], w2a.T, w2b.T)


# speedup vs baseline: 8.6424x; 1.4296x over previous
import functools

import jax
import jax.numpy as jnp
from jax import lax
from jax.experimental import pallas as pl
from jax.experimental.pallas import tpu as pltpu
from jax.experimental.pallas import tpu_sc as plsc

B, L = 4096, 50
EMB = 96
DIM = EMB * 2

EMB_P = 128
NC, NS = 2, 16
NW = NC * NS
BPW = B // NW
G = 8
CHUNKS = BPW // G
ROWS = G * L
SUB = 80
NSUB = ROWS // SUB
CH = EMB // 16


def _bag_pool_sc(idx_flat, table):
    mesh = plsc.VectorSubcoreMesh(core_axis_name="c", subcore_axis_name="s")

    @functools.partial(
        pl.kernel,
        mesh=mesh,
        out_type=jax.ShapeDtypeStruct((B, EMB), jnp.float32),
        scratch_types=[
            pltpu.VMEM((ROWS,), jnp.int32),
            pltpu.VMEM((ROWS,), jnp.int32),
            pltpu.VMEM((ROWS, EMB_P), jnp.float32),
            pltpu.VMEM((ROWS, EMB_P), jnp.float32),
            pltpu.VMEM((BPW, EMB), jnp.float32),
            pltpu.SemaphoreType.DMA,
            pltpu.SemaphoreType.DMA,
        ],
    )
    def kern(idx_hbm, tab_hbm, out_hbm, idx0, idx1, buf0, buf1, out_acc,
             sem0, sem1):
        w = lax.axis_index("s") * NC + lax.axis_index("c")
        base_i = w * (BPW * L)

        def fire(k, idxbuf, rowbuf, sem):
            pltpu.sync_copy(idx_hbm.at[pl.ds(base_i + k * ROWS, ROWS)],
                            idxbuf)
            for s in range(NSUB):
                sl = pl.ds(s * SUB, SUB)
                pltpu.async_copy(tab_hbm.at[idxbuf.at[sl]], rowbuf.at[sl],
                                 sem)

        def drain(rowbuf, sem):
            pltpu.make_async_copy(tab_hbm.at[pl.ds(0, ROWS)], rowbuf,
                                  sem).wait()

        def accum(k, rowbuf):
            def bag(j, _):
                base = j * L
                accs = [rowbuf[base, pl.ds(c * 16, 16)] for c in range(CH)]
                for r in range(1, L):
                    for c in range(CH):
                        accs[c] = accs[c] + rowbuf[base + r,
                                                   pl.ds(c * 16, 16)]
                row = k * G + j
                for c in range(CH):
                    out_acc[row, pl.ds(c * 16, 16)] = accs[c]
                return ()
            lax.fori_loop(0, G, bag, (), unroll=False)

        fire(0, idx0, buf0, sem0)
        fire(1, idx1, buf1, sem1)

        def body(m, _):
            k = 2 * m
            drain(buf0, sem0)
            accum(k, buf0)
            fire(k + 2, idx0, buf0, sem0)
            drain(buf1, sem1)
            accum(k + 1, buf1)
            fire(k + 3, idx1, buf1, sem1)
            return ()

        lax.fori_loop(0, CHUNKS // 2 - 1, body, (), unroll=False)
        drain(buf0, sem0)
        accum(CHUNKS - 2, buf0)
        drain(buf1, sem1)
        accum(CHUNKS - 1, buf1)
        pltpu.sync_copy(out_acc, out_hbm.at[pl.ds(w * BPW, BPW)])

    return kern(idx_flat, table)


def _pad_body(at_ref, o_ref):
    pad = ((0, 0), (0, EMB_P - EMB))
    o_ref[...] = jnp.pad(at_ref[...].T, pad)


def _pad_table_tc(table_t):
    n = table_t.shape[1]
    bn = 4096
    grid = (n + bn - 1) // bn
    return pl.pallas_call(
        _pad_body,
        grid=(grid,),
        in_specs=[pl.BlockSpec((EMB, bn), lambda i: (0, i))],
        out_specs=pl.BlockSpec((bn, EMB_P), lambda i: (i, 0)),
        out_shape=jax.ShapeDtypeStruct((n, EMB_P), jnp.float32),
    )(table_t)


def _mlp_body(xj_ref, xr_ref, w1a_ref, w1b_ref, w2a_ref, w2b_ref, o_ref):
    h = jnp.dot(xj_ref[...], w1a_ref[...],
                preferred_element_type=jnp.float32)
    h += jnp.dot(xr_ref[...], w1b_ref[...],
                 preferred_element_type=jnp.float32)
    h = jnp.maximum(h, 0.0)
    h2 = jnp.maximum(
        jnp.dot(h, w2a_ref[...], preferred_element_type=jnp.float32), 0.0)
    z = jnp.dot(h2, w2b_ref[...], preferred_element_type=jnp.float32)
    o_ref[...] = 1.0 / (1.0 + jnp.exp(-z))


def _mlp_tc(xj, xr, w1at, w1bt, w2at, w2bt):
    return pl.pallas_call(
        _mlp_body,
        out_shape=jax.ShapeDtypeStruct((B, 3), jnp.float32),
    )(xj, xr, w1at, w1bt, w2at, w2bt)


def kernel(jd, resume, position, job_emb, geek_emb,
           W1, W21a, W21b, W22a, W22b, W23a, W23b):
    del position
    jd_flat = jnp.reshape(jd, (B * L,))
    rs_flat = jnp.reshape(resume, (B * L,))
    job_pad = _pad_table_tc(job_emb.T)
    xj = _bag_pool_sc(jd_flat, job_pad)
    geek_pad = _pad_table_tc(geek_emb.T)
    xr = _bag_pool_sc(rs_flat, geek_pad)

    H = DIM // 6
    w2a = jnp.zeros((EMB, EMB), jnp.float32)
    w2a = w2a.at[0:H, 0:64].set(W21a)
    w2a = w2a.at[H:2 * H, 32:96].set(W22a)
    w2a = w2a.at[2 * H:3 * H, :].set(W23a)
    w2b = jnp.zeros((3, EMB), jnp.float32)
    w2b = w2b.at[0, 0:H].set(W21b[0])
    w2b = w2b.at[1, H:2 * H].set(W22b[0])
    w2b = w2b.at[2, 2 * H:3 * H].set(W23b[0])

    w1t = W1.T
    return _mlp_tc(xj, xr, w1t[:EMB], w1t[EMB:], w2a.T, w2b.T)


# speedup vs baseline: 9.0102x; 1.0426x over previous
import functools

import jax
import jax.numpy as jnp
from jax import lax
from jax.experimental import pallas as pl
from jax.experimental.pallas import tpu as pltpu
from jax.experimental.pallas import tpu_sc as plsc

B, L = 4096, 50
EMB = 96
DIM = EMB * 2

EMB_P = 128
NC, NS = 2, 16
NW = NC * NS
BPW = B // NW
G = 8
CHUNKS = BPW // G
ROWS = G * L
SUB = 80
NSUB = ROWS // SUB
CH = EMB // 16


def _bag_pool_sc(idx_flat, table):
    mesh = plsc.VectorSubcoreMesh(core_axis_name="c", subcore_axis_name="s")

    @functools.partial(
        pl.kernel,
        mesh=mesh,
        out_type=jax.ShapeDtypeStruct((B, EMB), jnp.float32),
        scratch_types=[
            pltpu.VMEM((ROWS,), jnp.int32),
            pltpu.VMEM((ROWS,), jnp.int32),
            pltpu.VMEM((ROWS, EMB_P), jnp.float32),
            pltpu.VMEM((ROWS, EMB_P), jnp.float32),
            pltpu.VMEM((BPW, EMB), jnp.float32),
            pltpu.SemaphoreType.DMA,
            pltpu.SemaphoreType.DMA,
            pltpu.SemaphoreType.DMA,
            pltpu.SemaphoreType.DMA,
        ],
    )
    def kern(idx_hbm, tab_hbm, out_hbm, idx0, idx1, buf0, buf1, out_acc,
             sem0, sem1, isem0, isem1):
        w = lax.axis_index("s") * NC + lax.axis_index("c")
        base_i = w * (BPW * L)

        def fire_idx(k, idxbuf, isem):
            pltpu.async_copy(idx_hbm.at[pl.ds(base_i + k * ROWS, ROWS)],
                             idxbuf, isem)

        def wait_idx(idxbuf, isem):
            pltpu.make_async_copy(idx_hbm.at[pl.ds(0, ROWS)], idxbuf,
                                  isem).wait()

        def fire(idxbuf, rowbuf, sem):
            for s in range(NSUB):
                sl = pl.ds(s * SUB, SUB)
                pltpu.async_copy(tab_hbm.at[idxbuf.at[sl]], rowbuf.at[sl],
                                 sem)

        def drain(rowbuf, sem):
            pltpu.make_async_copy(tab_hbm.at[pl.ds(0, ROWS)], rowbuf,
                                  sem).wait()

        def accum(k, rowbuf):
            def bag(j, _):
                base = j * L
                accs = [rowbuf[base, pl.ds(c * 16, 16)] for c in range(CH)]
                for r in range(1, L):
                    for c in range(CH):
                        accs[c] = accs[c] + rowbuf[base + r,
                                                   pl.ds(c * 16, 16)]
                row = k * G + j
                for c in range(CH):
                    out_acc[row, pl.ds(c * 16, 16)] = accs[c]
                return ()
            lax.fori_loop(0, G, bag, (), unroll=False)

        pltpu.sync_copy(idx_hbm.at[pl.ds(base_i, ROWS)], idx0)
        fire(idx0, buf0, sem0)
        pltpu.sync_copy(idx_hbm.at[pl.ds(base_i + ROWS, ROWS)], idx1)
        fire(idx1, buf1, sem1)

        def body(m, _):
            k = 2 * m
            drain(buf0, sem0)
            fire_idx(k + 2, idx0, isem0)
            accum(k, buf0)
            wait_idx(idx0, isem0)
            fire(idx0, buf0, sem0)
            drain(buf1, sem1)
            fire_idx(k + 3, idx1, isem1)
            accum(k + 1, buf1)
            wait_idx(idx1, isem1)
            fire(idx1, buf1, sem1)
            return ()

        lax.fori_loop(0, CHUNKS // 2 - 1, body, (), unroll=False)
        drain(buf0, sem0)
        accum(CHUNKS - 2, buf0)
        drain(buf1, sem1)
        accum(CHUNKS - 1, buf1)
        pltpu.sync_copy(out_acc, out_hbm.at[pl.ds(w * BPW, BPW)])

    return kern(idx_flat, table)


def _pad_body(at_ref, o_ref):
    pad = ((0, 0), (0, EMB_P - EMB))
    o_ref[...] = jnp.pad(at_ref[...].T, pad)


def _pad_table_tc(table_t):
    n = table_t.shape[1]
    bn = 4096
    grid = (n + bn - 1) // bn
    return pl.pallas_call(
        _pad_body,
        grid=(grid,),
        in_specs=[pl.BlockSpec((EMB, bn), lambda i: (0, i))],
        out_specs=pl.BlockSpec((bn, EMB_P), lambda i: (i, 0)),
        out_shape=jax.ShapeDtypeStruct((n, EMB_P), jnp.float32),
    )(table_t)


def _mlp_body(xj_ref, xr_ref, w1a_ref, w1b_ref, w2a_ref, w2b_ref, o_ref):
    h = jnp.dot(xj_ref[...], w1a_ref[...],
                preferred_element_type=jnp.float32)
    h += jnp.dot(xr_ref[...], w1b_ref[...],
                 preferred_element_type=jnp.float32)
    h = jnp.maximum(h, 0.0)
    h2 = jnp.maximum(
        jnp.dot(h, w2a_ref[...], preferred_element_type=jnp.float32), 0.0)
    z = jnp.dot(h2, w2b_ref[...], preferred_element_type=jnp.float32)
    o_ref[...] = 1.0 / (1.0 + jnp.exp(-z))


def _mlp_tc(xj, xr, w1at, w1bt, w2at, w2bt):
    return pl.pallas_call(
        _mlp_body,
        out_shape=jax.ShapeDtypeStruct((B, 3), jnp.float32),
    )(xj, xr, w1at, w1bt, w2at, w2bt)


def kernel(jd, resume, position, job_emb, geek_emb,
           W1, W21a, W21b, W22a, W22b, W23a, W23b):
    del position
    jd_flat = jnp.reshape(jd, (B * L,))
    rs_flat = jnp.reshape(resume, (B * L,))
    job_pad = _pad_table_tc(job_emb.T)
    xj = _bag_pool_sc(jd_flat, job_pad)
    geek_pad = _pad_table_tc(geek_emb.T)
    xr = _bag_pool_sc(rs_flat, geek_pad)

    H = DIM // 6
    w2a = jnp.zeros((EMB, EMB), jnp.float32)
    w2a = w2a.at[0:H, 0:64].set(W21a)
    w2a = w2a.at[H:2 * H, 32:96].set(W22a)
    w2a = w2a.at[2 * H:3 * H, :].set(W23a)
    w2b = jnp.zeros((3, EMB), jnp.float32)
    w2b = w2b.at[0, 0:H].set(W21b[0])
    w2b = w2b.at[1, H:2 * H].set(W22b[0])
    w2b = w2b.at[2, 2 * H:3 * H].set(W23b[0])

    w1t = W1.T
    return _mlp_tc(xj, xr, w1t[:EMB], w1t[EMB:], w2a.T, w2b.T)
